# Initial kernel scaffold; baseline (speedup 1.0000x reference)
import functools

import jax
import jax.numpy as jnp
from jax import lax
from jax.experimental import pallas as pl
from jax.experimental.pallas import tpu as pltpu
from jax.experimental.pallas import tpu_sc as plsc

N = 10000
E = 160000
NC = 2
NS = 16
NW = NC * NS
CH = 128
EPW = 5120
E_PAD = NW * EPW
NCH = EPW // CH
N_PAD = 10240
NPT = N_PAD // NS

_MESH = plsc.VectorSubcoreMesh(
    core_axis_name="c", subcore_axis_name="s", num_cores=NC, num_subcores=NS)


def _worker_base():
    c = lax.axis_index("c")
    s = lax.axis_index("s")
    wid = s * NC + c
    return c, s, wid * EPW


def _gather_loop(idx_hbm, table_hbm, out_hbm, idx_v, buf_v, sem, base):
    def body(j, carry):
        off = pl.multiple_of(base + j * CH, CH)
        pltpu.sync_copy(idx_hbm.at[pl.ds(off, CH)], idx_v)
        pltpu.async_copy(table_hbm.at[idx_v], buf_v, sem).wait()
        pltpu.sync_copy(buf_v, out_hbm.at[pl.ds(off, CH)])
        return carry
    lax.fori_loop(0, NCH, body, 0)


def _scatter_loop(idx_hbm, vals_hbm, acc_shared, idx_v, buf_v, base):
    def body(j, carry):
        off = pl.multiple_of(base + j * CH, CH)
        pltpu.sync_copy(idx_hbm.at[pl.ds(off, CH)], idx_v)
        pltpu.sync_copy(vals_hbm.at[pl.ds(off, CH)], buf_v)
        pltpu.sync_copy(buf_v, acc_shared.at[idx_v], add=True)
        return carry
    lax.fori_loop(0, NCH, body, 0)


def _acc_init(zeros_hbm, acc_shared, s):
    r = pl.multiple_of(s * NPT, NPT)
    pltpu.sync_copy(zeros_hbm.at[pl.ds(r, NPT)], acc_shared.at[pl.ds(r, NPT)])


def _acc_dump(acc_shared, out_hbm, c, s):
    r = pl.multiple_of(s * NPT, NPT)
    pltpu.sync_copy(acc_shared.at[pl.ds(r, NPT)], out_hbm.at[c, pl.ds(r, NPT)])


def _sc1_body(srcm, xm16, col, ew16, zeros16,
              xs16, degp, acc, idx_v, bufg, bufs, sem):
    c, s, base = _worker_base()
    _acc_init(zeros16, acc, s)
    _gather_loop(srcm, xm16, xs16, idx_v, bufg, sem, base)
    plsc.subcore_barrier()
    _scatter_loop(col, ew16, acc, idx_v, bufs, base)
    plsc.subcore_barrier()
    _acc_dump(acc, degp, c, s)


_sc1 = functools.partial(
    pl.kernel, _sc1_body,
    out_type=[jax.ShapeDtypeStruct((E_PAD, 16), jnp.float32),
              jax.ShapeDtypeStruct((NC, N_PAD, 16), jnp.float32)],
    mesh=_MESH,
    scratch_types=[pltpu.VMEM_SHARED((N_PAD, 16), jnp.float32),
                   pltpu.VMEM((CH,), jnp.int32),
                   pltpu.VMEM((CH, 16), jnp.float32),
                   pltpu.VMEM((CH, 16), jnp.float32),
                   pltpu.SemaphoreType.DMA])


def _sc_gs_body(gidx, table, sidx, vals, zeros64,
                gout, accp, acc, idx_v, bufg, bufs, sem):
    c, s, base = _worker_base()
    _acc_init(zeros64, acc, s)
    _gather_loop(gidx, table, gout, idx_v, bufg, sem, base)
    plsc.subcore_barrier()
    _scatter_loop(sidx, vals, acc, idx_v, bufs, base)
    plsc.subcore_barrier()
    _acc_dump(acc, accp, c, s)


_sc_gs = functools.partial(
    pl.kernel, _sc_gs_body,
    out_type=[jax.ShapeDtypeStruct((E_PAD, 64), jnp.float32),
              jax.ShapeDtypeStruct((NC, N_PAD, 64), jnp.float32)],
    mesh=_MESH,
    scratch_types=[pltpu.VMEM_SHARED((N_PAD, 64), jnp.float32),
                   pltpu.VMEM((CH,), jnp.int32),
                   pltpu.VMEM((CH, 64), jnp.float32),
                   pltpu.VMEM((CH, 64), jnp.float32),
                   pltpu.SemaphoreType.DMA])


def _sc_s_body(sidx, vals, zeros64, accp, acc, idx_v, bufs, sem):
    c, s, base = _worker_base()
    _acc_init(zeros64, acc, s)
    plsc.subcore_barrier()
    _scatter_loop(sidx, vals, acc, idx_v, bufs, base)
    plsc.subcore_barrier()
    _acc_dump(acc, accp, c, s)


_sc_s = functools.partial(
    pl.kernel, _sc_s_body,
    out_type=[jax.ShapeDtypeStruct((NC, N_PAD, 64), jnp.float32)],
    mesh=_MESH,
    scratch_types=[pltpu.VMEM_SHARED((N_PAD, 64), jnp.float32),
                   pltpu.VMEM((CH,), jnp.int32),
                   pltpu.VMEM((CH, 64), jnp.float32),
                   pltpu.SemaphoreType.DMA])


RB = 1000
EB = 512
_NG = N // RB
_EG = E_PAD // EB


def _tc_nodeA_body(x_ref, w_ref, degp_ref, y_ref, dinv_ref):
    deg = degp_ref[0, :, 0:1] + degp_ref[1, :, 0:1] + 1.0
    dinv = lax.rsqrt(deg)
    xw = jnp.dot(x_ref[...], w_ref[...], preferred_element_type=jnp.float32)
    y_ref[...] = dinv * xw
    dinv_ref[...] = dinv


def _tc_msg1_body(ea_ref, xs_ref, v0_ref, v1_ref, nb_ref, out_ref):
    a = ea_ref[:, 0:1]
    b = ea_ref[:, 1:2]
    xs = xs_ref[...]
    acc = jnp.zeros((EB, 64), jnp.float32)
    for i in range(8):
        w = jnp.maximum(a * v0_ref[i:i + 1, :] + b * v1_ref[i:i + 1, :]
                        + nb_ref[i:i + 1, :], 0.0)
        acc = acc + xs[:, i:i + 1] * w
    out_ref[...] = acc


def _tc_h1_body(aggp_ref, xm_ref, r1_ref, b1_ref, out_ref):
    xr = jnp.dot(xm_ref[...], r1_ref[...], preferred_element_type=jnp.float32)
    out_ref[...] = aggp_ref[0] + aggp_ref[1] + xr + b1_ref[...]


def _tc_gmsg_body(ew_ref, g_ref, out_ref):
    out_ref[...] = ew_ref[...] * g_ref[...]


def _tc_msg2_body(ea_ref, hs_ref, w0_ref, w1_ref, nb_ref, out_ref):
    a = ea_ref[:, 0:1]
    b = ea_ref[:, 1:2]
    hs = hs_ref[...]
    acc = jnp.zeros((EB, 64), jnp.float32)
    for i in range(64):
        w = jnp.maximum(a * w0_ref[i:i + 1, :] + b * w1_ref[i:i + 1, :]
                        + nb_ref[i:i + 1, :], 0.0)
        acc = acc + hs[:, i:i + 1] * w
    out_ref[...] = acc


def _tc_final_body(gaccp_ref, agg2p_ref, dinv_ref, y_ref, h1_ref, hid_ref,
                   x_ref, r2_ref, b2_ref, gb_ref, hA_ref, hB_ref, hC_ref,
                   hD_ref, hb_ref, sA_ref, sB_ref, sC_ref, sb_ref,
                   sc_ref, hs_ref):
    dinv = dinv_ref[...]
    epi = dinv * (gaccp_ref[0] + gaccp_ref[1]) + dinv * y_ref[...] + gb_ref[...]
    info = (agg2p_ref[0] + agg2p_ref[1]
            + jnp.dot(h1_ref[...], r2_ref[...],
                      preferred_element_type=jnp.float32)
            + b2_ref[...])
    t = (jnp.dot(epi, hA_ref[...], preferred_element_type=jnp.float32)
         + jnp.dot(info, hB_ref[...], preferred_element_type=jnp.float32)
         + jnp.dot(hid_ref[...], hC_ref[...], preferred_element_type=jnp.float32)
         + jnp.dot(x_ref[...], hD_ref[...], preferred_element_type=jnp.float32)
         + hb_ref[...])
    hs = jnp.maximum(t, 0.0)
    nrm = jnp.sqrt(jnp.sum(hs * hs, axis=1, keepdims=True))
    hsn = hs / jnp.maximum(nrm, 1e-12)
    s = (jnp.sum(hid_ref[...] * sA_ref[...], axis=1, keepdims=True)
         + jnp.sum(hsn * sB_ref[...], axis=1, keepdims=True)
         + jnp.sum(x_ref[...] * sC_ref[...], axis=1, keepdims=True)
         + sb_ref[...])
    sc_ref[...] = jnp.maximum(s, 0.0)
    hs_ref[...] = hsn


def _full(shape):
    return pl.BlockSpec(shape, lambda i: (0,) * len(shape))


def _rows(shape):
    return pl.BlockSpec(shape, lambda i: (i,) + (0,) * (len(shape) - 1))


def _mid(shape):
    return pl.BlockSpec(shape, lambda i: (0, i) + (0,) * (len(shape) - 2))


def kernel(x, edge_index, edge_weight, x_multi, edge_index_multi,
           edge_attr_multi, hidden_states, gcn_W, gcn_b, nn1_W, nn1_b,
           root1, b1, nn2_W, nn2_b, root2, b2, h_W, h_b, s_W, s_b):
    f32 = jnp.float32
    pe = E_PAD - E
    row = jnp.pad(edge_index[0], (0, pe))
    col = jnp.pad(edge_index[1], (0, pe), constant_values=N)
    srcm = jnp.pad(edge_index_multi[0], (0, pe))
    dstm = jnp.pad(edge_index_multi[1], (0, pe), constant_values=N)
    ew = jnp.pad(edge_weight, (0, pe))
    ea = jnp.pad(edge_attr_multi, ((0, pe), (0, 0)))
    ew16 = jnp.pad(ew[:, None], ((0, 0), (0, 15)))
    xm16 = jnp.pad(x_multi, ((0, 0), (0, 8)))
    zeros16 = jnp.zeros((N_PAD, 16), f32)
    zeros64 = jnp.zeros((N_PAD, 64), f32)

    v0 = nn1_W[0].reshape(8, 64)
    v1 = nn1_W[1].reshape(8, 64)
    nb1 = nn1_b.reshape(8, 64)
    w0 = nn2_W[0].reshape(64, 64)
    w1 = nn2_W[1].reshape(64, 64)
    nb2 = nn2_b.reshape(64, 64)
    b1r = b1.reshape(1, 64)
    b2r = b2.reshape(1, 64)
    gbr = gcn_b.reshape(1, 64)
    hA = h_W[0:64]
    hB = h_W[64:128]
    hC = h_W[128:192]
    hD = h_W[192:200]
    hbr = h_b.reshape(1, 64)
    sA = s_W[0:64].reshape(1, 64)
    sB = s_W[64:128].reshape(1, 64)
    sC = s_W[128:136].reshape(1, 8)
    sbr = s_b.reshape(1, 1)

    xs16, degp = _sc1()(srcm, xm16, col, ew16, zeros16)

    y, dinv = pl.pallas_call(
        _tc_nodeA_body,
        grid=(_NG,),
        in_specs=[_rows((RB, 8)), _full((8, 64)), _mid((NC, RB, 16))],
        out_specs=[_rows((RB, 64)), _rows((RB, 1))],
        out_shape=[jax.ShapeDtypeStruct((N, 64), f32),
                   jax.ShapeDtypeStruct((N, 1), f32)],
    )(x, gcn_W, degp)

    msg1 = pl.pallas_call(
        _tc_msg1_body,
        grid=(_EG,),
        in_specs=[_rows((EB, 2)), _rows((EB, 16)), _full((8, 64)),
                  _full((8, 64)), _full((8, 64))],
        out_specs=_rows((EB, 64)),
        out_shape=jax.ShapeDtypeStruct((E_PAD, 64), f32),
    )(ea, xs16, v0, v1, nb1)

    g64, agg1p = _sc_gs()(row, y, dstm, msg1, zeros64)

    h1 = pl.pallas_call(
        _tc_h1_body,
        grid=(_NG,),
        in_specs=[_mid((NC, RB, 64)), _rows((RB, 8)), _full((8, 64)),
                  _full((1, 64))],
        out_specs=_rows((RB, 64)),
        out_shape=jax.ShapeDtypeStruct((N, 64), f32),
    )(agg1p, x_multi, root1, b1r)

    gmsg = pl.pallas_call(
        _tc_gmsg_body,
        grid=(_EG,),
        in_specs=[_rows((EB, 1)), _rows((EB, 64))],
        out_specs=_rows((EB, 64)),
        out_shape=jax.ShapeDtypeStruct((E_PAD, 64), f32),
    )(ew[:, None], g64)

    hs64, gaccp = _sc_gs()(srcm, h1, col, gmsg, zeros64)

    msg2 = pl.pallas_call(
        _tc_msg2_body,
        grid=(_EG,),
        in_specs=[_rows((EB, 2)), _rows((EB, 64)), _full((64, 64)),
                  _full((64, 64)), _full((64, 64))],
        out_specs=_rows((EB, 64)),
        out_shape=jax.ShapeDtypeStruct((E_PAD, 64), f32),
    )(ea, hs64, w0, w1, nb2)

    agg2p = _sc_s()(dstm, msg2, zeros64)

    sc, hs = pl.pallas_call(
        _tc_final_body,
        grid=(_NG,),
        in_specs=[_mid((NC, RB, 64)), _mid((NC, RB, 64)), _rows((RB, 1)),
                  _rows((RB, 64)), _rows((RB, 64)), _rows((RB, 64)),
                  _rows((RB, 8)), _full((64, 64)), _full((1, 64)),
                  _full((1, 64)), _full((64, 64)), _full((64, 64)),
                  _full((64, 64)), _full((8, 64)), _full((1, 64)),
                  _full((1, 64)), _full((1, 64)), _full((1, 8)),
                  _full((1, 1))],
        out_specs=[_rows((RB, 1)), _rows((RB, 64))],
        out_shape=[jax.ShapeDtypeStruct((N, 1), f32),
                   jax.ShapeDtypeStruct((N, 64), f32)],
    )(gaccp, agg2p, dinv, y, h1, hidden_states, x, root2, b2r, gbr,
      hA, hB, hC, hD, hbr, sA, sB, sC, sbr)

    return (sc, hs)


# speedup vs baseline: 1.4948x; 1.4948x over previous
import functools

import jax
import jax.numpy as jnp
from jax import lax
from jax.experimental import pallas as pl
from jax.experimental.pallas import tpu as pltpu
from jax.experimental.pallas import tpu_sc as plsc

N = 10000
E = 160000
NC = 2
NS = 16
NW = NC * NS
CH = 128
EPW = 5120
E_PAD = NW * EPW
NCH = EPW // CH
N_PAD = 10240
NPT = N_PAD // NS

@functools.lru_cache(maxsize=None)
def _mesh():
    return plsc.VectorSubcoreMesh(
        core_axis_name="c", subcore_axis_name="s",
        num_cores=NC, num_subcores=NS)


def _worker_base():
    c = lax.axis_index("c")
    s = lax.axis_index("s")
    wid = s * NC + c
    return c, s, wid * EPW


def _gather_loop(idx_hbm, table_hbm, out_hbm, idx_v, buf_v, sem, base):
    def body(j, carry):
        off = pl.multiple_of(base + j * CH, CH)
        pltpu.sync_copy(idx_hbm.at[pl.ds(off, CH)], idx_v)
        pltpu.async_copy(table_hbm.at[idx_v], buf_v, sem).wait()
        pltpu.sync_copy(buf_v, out_hbm.at[pl.ds(off, CH)])
        return carry
    lax.fori_loop(0, NCH, body, 0)


def _scatter_loop(idx_hbm, vals_hbm, acc_shared, idx_v, buf_v, base):
    def body(j, carry):
        off = pl.multiple_of(base + j * CH, CH)
        pltpu.sync_copy(idx_hbm.at[pl.ds(off, CH)], idx_v)
        pltpu.sync_copy(vals_hbm.at[pl.ds(off, CH)], buf_v)
        pltpu.sync_copy(buf_v, acc_shared.at[idx_v], add=True)
        return carry
    lax.fori_loop(0, NCH, body, 0)


def _acc_init(zeros_hbm, acc_shared, s):
    r = pl.multiple_of(s * NPT, NPT)
    pltpu.sync_copy(zeros_hbm.at[pl.ds(r, NPT)], acc_shared.at[pl.ds(r, NPT)])


def _acc_dump(acc_shared, out_hbm, c, s):
    r = pl.multiple_of(s * NPT, NPT)
    pltpu.sync_copy(acc_shared.at[pl.ds(r, NPT)], out_hbm.at[c, pl.ds(r, NPT)])


def _sc1_body(srcm, xm16, col, ew16, zeros16,
              xs16, degp, acc, idx_v, bufg, bufs, sem):
    c, s, base = _worker_base()
    _acc_init(zeros16, acc, s)
    _gather_loop(srcm, xm16, xs16, idx_v, bufg, sem, base)
    plsc.subcore_barrier()
    _scatter_loop(col, ew16, acc, idx_v, bufs, base)
    plsc.subcore_barrier()
    _acc_dump(acc, degp, c, s)


@functools.lru_cache(maxsize=None)
def _sc1():
    return pl.kernel(
        _sc1_body,
        out_type=[jax.ShapeDtypeStruct((E_PAD, 16), jnp.float32),
                  jax.ShapeDtypeStruct((NC, N_PAD, 16), jnp.float32)],
        mesh=_mesh(),
        compiler_params=pltpu.CompilerParams(use_tc_tiling_on_sc=False),
        scratch_types=[pltpu.VMEM_SHARED((N_PAD, 16), jnp.float32),
                       pltpu.VMEM((CH,), jnp.int32),
                       pltpu.VMEM((CH, 16), jnp.float32),
                       pltpu.VMEM((CH, 16), jnp.float32),
                       pltpu.SemaphoreType.DMA])


def _sc_gs_body(gidx, table, sidx, vals, zeros64,
                gout, accp, acc, idx_v, bufg, bufs, sem):
    c, s, base = _worker_base()
    _acc_init(zeros64, acc, s)
    _gather_loop(gidx, table, gout, idx_v, bufg, sem, base)
    plsc.subcore_barrier()
    _scatter_loop(sidx, vals, acc, idx_v, bufs, base)
    plsc.subcore_barrier()
    _acc_dump(acc, accp, c, s)


@functools.lru_cache(maxsize=None)
def _sc_gs():
    return pl.kernel(
        _sc_gs_body,
        out_type=[jax.ShapeDtypeStruct((E_PAD, 64), jnp.float32),
                  jax.ShapeDtypeStruct((NC, N_PAD, 64), jnp.float32)],
        mesh=_mesh(),
        compiler_params=pltpu.CompilerParams(use_tc_tiling_on_sc=False),
        scratch_types=[pltpu.VMEM_SHARED((N_PAD, 64), jnp.float32),
                       pltpu.VMEM((CH,), jnp.int32),
                       pltpu.VMEM((CH, 64), jnp.float32),
                       pltpu.VMEM((CH, 64), jnp.float32),
                       pltpu.SemaphoreType.DMA])


def _sc_s_body(sidx, vals, zeros64, accp, acc, idx_v, bufs, sem):
    c, s, base = _worker_base()
    _acc_init(zeros64, acc, s)
    plsc.subcore_barrier()
    _scatter_loop(sidx, vals, acc, idx_v, bufs, base)
    plsc.subcore_barrier()
    _acc_dump(acc, accp, c, s)


@functools.lru_cache(maxsize=None)
def _sc_s():
    return pl.kernel(
        _sc_s_body,
        out_type=[jax.ShapeDtypeStruct((NC, N_PAD, 64), jnp.float32)],
        mesh=_mesh(),
        compiler_params=pltpu.CompilerParams(use_tc_tiling_on_sc=False),
        scratch_types=[pltpu.VMEM_SHARED((N_PAD, 64), jnp.float32),
                       pltpu.VMEM((CH,), jnp.int32),
                       pltpu.VMEM((CH, 64), jnp.float32),
                       pltpu.SemaphoreType.DMA])


RB = 1000
EB = 512
_NG = N // RB
_EG = E_PAD // EB


def _tc_nodeA_body(x_ref, w_ref, degp_ref, y_ref, dinv_ref):
    deg = degp_ref[0, :, 0:1] + degp_ref[1, :, 0:1] + 1.0
    dinv = lax.rsqrt(deg)
    xw = jnp.dot(x_ref[...], w_ref[...], preferred_element_type=jnp.float32)
    y_ref[...] = dinv * xw
    dinv_ref[...] = dinv


def _tc_msg1_body(ea_ref, xs_ref, v0_ref, v1_ref, nb_ref, out_ref):
    a = ea_ref[:, 0:1]
    b = ea_ref[:, 1:2]
    xs = xs_ref[...]
    acc = jnp.zeros((EB, 64), jnp.float32)
    for i in range(8):
        w = jnp.maximum(a * v0_ref[i:i + 1, :] + b * v1_ref[i:i + 1, :]
                        + nb_ref[i:i + 1, :], 0.0)
        acc = acc + xs[:, i:i + 1] * w
    out_ref[...] = acc


def _tc_h1_body(aggp_ref, xm_ref, r1_ref, b1_ref, out_ref):
    xr = jnp.dot(xm_ref[...], r1_ref[...], preferred_element_type=jnp.float32)
    out_ref[...] = aggp_ref[0] + aggp_ref[1] + xr + b1_ref[...]


def _tc_gmsg_body(ew_ref, g_ref, out_ref):
    out_ref[...] = ew_ref[...] * g_ref[...]


def _tc_msg2_body(ea_ref, hs_ref, w0_ref, w1_ref, nb_ref, out_ref):
    a = ea_ref[:, 0:1]
    b = ea_ref[:, 1:2]
    hs = hs_ref[...]
    acc = jnp.zeros((EB, 64), jnp.float32)
    for i in range(64):
        w = jnp.maximum(a * w0_ref[i:i + 1, :] + b * w1_ref[i:i + 1, :]
                        + nb_ref[i:i + 1, :], 0.0)
        acc = acc + hs[:, i:i + 1] * w
    out_ref[...] = acc


def _tc_final_body(gaccp_ref, agg2p_ref, dinv_ref, y_ref, h1_ref, hid_ref,
                   x_ref, r2_ref, b2_ref, gb_ref, hA_ref, hB_ref, hC_ref,
                   hD_ref, hb_ref, sA_ref, sB_ref, sC_ref, sb_ref,
                   sc_ref, hs_ref):
    dinv = dinv_ref[...]
    epi = dinv * (gaccp_ref[0] + gaccp_ref[1]) + dinv * y_ref[...] + gb_ref[...]
    info = (agg2p_ref[0] + agg2p_ref[1]
            + jnp.dot(h1_ref[...], r2_ref[...],
                      preferred_element_type=jnp.float32)
            + b2_ref[...])
    t = (jnp.dot(epi, hA_ref[...], preferred_element_type=jnp.float32)
         + jnp.dot(info, hB_ref[...], preferred_element_type=jnp.float32)
         + jnp.dot(hid_ref[...], hC_ref[...], preferred_element_type=jnp.float32)
         + jnp.dot(x_ref[...], hD_ref[...], preferred_element_type=jnp.float32)
         + hb_ref[...])
    hs = jnp.maximum(t, 0.0)
    nrm = jnp.sqrt(jnp.sum(hs * hs, axis=1, keepdims=True))
    hsn = hs / jnp.maximum(nrm, 1e-12)
    s = (jnp.sum(hid_ref[...] * sA_ref[...], axis=1, keepdims=True)
         + jnp.sum(hsn * sB_ref[...], axis=1, keepdims=True)
         + jnp.sum(x_ref[...] * sC_ref[...], axis=1, keepdims=True)
         + sb_ref[...])
    sc_ref[...] = jnp.maximum(s, 0.0)
    hs_ref[...] = hsn


def _full(shape):
    return pl.BlockSpec(shape, lambda i: (0,) * len(shape))


def _rows(shape):
    return pl.BlockSpec(shape, lambda i: (i,) + (0,) * (len(shape) - 1))


def _mid(shape):
    return pl.BlockSpec(shape, lambda i: (0, i) + (0,) * (len(shape) - 2))


def kernel(x, edge_index, edge_weight, x_multi, edge_index_multi,
           edge_attr_multi, hidden_states, gcn_W, gcn_b, nn1_W, nn1_b,
           root1, b1, nn2_W, nn2_b, root2, b2, h_W, h_b, s_W, s_b):
    f32 = jnp.float32
    pe = E_PAD - E
    row = jnp.pad(edge_index[0], (0, pe))
    col = jnp.pad(edge_index[1], (0, pe), constant_values=N)
    srcm = jnp.pad(edge_index_multi[0], (0, pe))
    dstm = jnp.pad(edge_index_multi[1], (0, pe), constant_values=N)
    ew = jnp.pad(edge_weight, (0, pe))
    ea = jnp.pad(edge_attr_multi, ((0, pe), (0, 0)))
    ew16 = jnp.pad(ew[:, None], ((0, 0), (0, 15)))
    xm16 = jnp.pad(x_multi, ((0, 0), (0, 8)))
    zeros16 = jnp.zeros((N_PAD, 16), f32)
    zeros64 = jnp.zeros((N_PAD, 64), f32)

    v0 = nn1_W[0].reshape(8, 64)
    v1 = nn1_W[1].reshape(8, 64)
    nb1 = nn1_b.reshape(8, 64)
    w0 = nn2_W[0].reshape(64, 64)
    w1 = nn2_W[1].reshape(64, 64)
    nb2 = nn2_b.reshape(64, 64)
    b1r = b1.reshape(1, 64)
    b2r = b2.reshape(1, 64)
    gbr = gcn_b.reshape(1, 64)
    hA = h_W[0:64]
    hB = h_W[64:128]
    hC = h_W[128:192]
    hD = h_W[192:200]
    hbr = h_b.reshape(1, 64)
    sA = s_W[0:64].reshape(1, 64)
    sB = s_W[64:128].reshape(1, 64)
    sC = s_W[128:136].reshape(1, 8)
    sbr = s_b.reshape(1, 1)

    xs16, degp = _sc1()(srcm, xm16, col, ew16, zeros16)

    y, dinv = pl.pallas_call(
        _tc_nodeA_body,
        grid=(_NG,),
        in_specs=[_rows((RB, 8)), _full((8, 64)), _mid((NC, RB, 16))],
        out_specs=[_rows((RB, 64)), _rows((RB, 1))],
        out_shape=[jax.ShapeDtypeStruct((N, 64), f32),
                   jax.ShapeDtypeStruct((N, 1), f32)],
    )(x, gcn_W, degp)

    msg1 = pl.pallas_call(
        _tc_msg1_body,
        grid=(_EG,),
        in_specs=[_rows((EB, 2)), _rows((EB, 16)), _full((8, 64)),
                  _full((8, 64)), _full((8, 64))],
        out_specs=_rows((EB, 64)),
        out_shape=jax.ShapeDtypeStruct((E_PAD, 64), f32),
    )(ea, xs16, v0, v1, nb1)

    g64, agg1p = _sc_gs()(row, y, dstm, msg1, zeros64)

    h1 = pl.pallas_call(
        _tc_h1_body,
        grid=(_NG,),
        in_specs=[_mid((NC, RB, 64)), _rows((RB, 8)), _full((8, 64)),
                  _full((1, 64))],
        out_specs=_rows((RB, 64)),
        out_shape=jax.ShapeDtypeStruct((N, 64), f32),
    )(agg1p, x_multi, root1, b1r)

    gmsg = pl.pallas_call(
        _tc_gmsg_body,
        grid=(_EG,),
        in_specs=[_rows((EB, 1)), _rows((EB, 64))],
        out_specs=_rows((EB, 64)),
        out_shape=jax.ShapeDtypeStruct((E_PAD, 64), f32),
    )(ew[:, None], g64)

    hs64, gaccp = _sc_gs()(srcm, h1, col, gmsg, zeros64)

    msg2 = pl.pallas_call(
        _tc_msg2_body,
        grid=(_EG,),
        in_specs=[_rows((EB, 2)), _rows((EB, 64)), _full((64, 64)),
                  _full((64, 64)), _full((64, 64))],
        out_specs=_rows((EB, 64)),
        out_shape=jax.ShapeDtypeStruct((E_PAD, 64), f32),
    )(ea, hs64, w0, w1, nb2)

    (agg2p,) = _sc_s()(dstm, msg2, zeros64)

    sc, hs = pl.pallas_call(
        _tc_final_body,
        grid=(_NG,),
        in_specs=[_mid((NC, RB, 64)), _mid((NC, RB, 64)), _rows((RB, 1)),
                  _rows((RB, 64)), _rows((RB, 64)), _rows((RB, 64)),
                  _rows((RB, 8)), _full((64, 64)), _full((1, 64)),
                  _full((1, 64)), _full((64, 64)), _full((64, 64)),
                  _full((64, 64)), _full((8, 64)), _full((1, 64)),
                  _full((1, 64)), _full((1, 64)), _full((1, 8)),
                  _full((1, 1))],
        out_specs=[_rows((RB, 1)), _rows((RB, 64))],
        out_shape=[jax.ShapeDtypeStruct((N, 1), f32),
                   jax.ShapeDtypeStruct((N, 64), f32)],
    )(gaccp, agg2p, dinv, y, h1, hidden_states, x, root2, b2r, gbr,
      hA, hB, hC, hD, hbr, sA, sB, sC, sbr)

    return (sc, hs)


# speedup vs baseline: 1.9724x; 1.3195x over previous
import functools

import jax
import jax.numpy as jnp
from jax import lax
from jax.experimental import pallas as pl
from jax.experimental.pallas import tpu as pltpu
from jax.experimental.pallas import tpu_sc as plsc

N = 10000
E = 160000
NC = 2
NS = 16
NW = NC * NS
CH = 128
EPW = 5120
E_PAD = NW * EPW
NCH = EPW // CH
N_PAD = 10240
NPT = N_PAD // NS

@functools.lru_cache(maxsize=None)
def _mesh():
    return plsc.VectorSubcoreMesh(
        core_axis_name="c", subcore_axis_name="s",
        num_cores=NC, num_subcores=NS)


def _worker_base():
    c = lax.axis_index("c")
    s = lax.axis_index("s")
    wid = s * NC + c
    return c, s, wid * EPW


def _gather_loop(idx_hbm, table_hbm, out_hbm, idx_v, buf_v, sem, base):
    def body(j, carry):
        off = pl.multiple_of(base + j * CH, CH)
        pltpu.sync_copy(idx_hbm.at[pl.ds(off, CH)], idx_v)
        pltpu.async_copy(table_hbm.at[idx_v], buf_v, sem).wait()
        pltpu.sync_copy(buf_v, out_hbm.at[pl.ds(off, CH)])
        return carry
    lax.fori_loop(0, NCH, body, 0)


def _scatter_loop(idx_hbm, vals_hbm, acc_shared, idx_v, buf_v, base):
    def body(j, carry):
        off = pl.multiple_of(base + j * CH, CH)
        pltpu.sync_copy(idx_hbm.at[pl.ds(off, CH)], idx_v)
        pltpu.sync_copy(vals_hbm.at[pl.ds(off, CH)], buf_v)
        pltpu.sync_copy(buf_v, acc_shared.at[idx_v], add=True)
        return carry
    lax.fori_loop(0, NCH, body, 0)


def _acc_init(zeros_hbm, acc_shared, s):
    r = pl.multiple_of(s * NPT, NPT)
    pltpu.sync_copy(zeros_hbm.at[pl.ds(r, NPT)], acc_shared.at[pl.ds(r, NPT)])


def _acc_dump(acc_shared, out_hbm, c, s):
    r = pl.multiple_of(s * NPT, NPT)
    pltpu.sync_copy(acc_shared.at[pl.ds(r, NPT)], out_hbm.at[c, pl.ds(r, NPT)])


def _sc1_body(srcm, xm16, col, ew16, zeros16,
              xs16, degp, acc, idx_v, bufg, bufs, sem):
    c, s, base = _worker_base()
    _acc_init(zeros16, acc, s)
    _gather_loop(srcm, xm16, xs16, idx_v, bufg, sem, base)
    plsc.subcore_barrier()
    _scatter_loop(col, ew16, acc, idx_v, bufs, base)
    plsc.subcore_barrier()
    _acc_dump(acc, degp, c, s)


@functools.lru_cache(maxsize=None)
def _sc1():
    return pl.kernel(
        _sc1_body,
        out_type=[jax.ShapeDtypeStruct((E_PAD, 16), jnp.float32),
                  jax.ShapeDtypeStruct((NC, N_PAD, 16), jnp.float32)],
        mesh=_mesh(),
        compiler_params=pltpu.CompilerParams(use_tc_tiling_on_sc=False),
        scratch_types=[pltpu.VMEM_SHARED((N_PAD, 16), jnp.float32),
                       pltpu.VMEM((CH,), jnp.int32),
                       pltpu.VMEM((CH, 16), jnp.float32),
                       pltpu.VMEM((CH, 16), jnp.float32),
                       pltpu.SemaphoreType.DMA])


def _sc_gs_body(gidx, table, sidx, vals, zeros64,
                gout, accp, acc, idx_v, bufg, bufs, sem):
    c, s, base = _worker_base()
    _acc_init(zeros64, acc, s)
    _gather_loop(gidx, table, gout, idx_v, bufg, sem, base)
    plsc.subcore_barrier()
    _scatter_loop(sidx, vals, acc, idx_v, bufs, base)
    plsc.subcore_barrier()
    _acc_dump(acc, accp, c, s)


@functools.lru_cache(maxsize=None)
def _sc_gs():
    return pl.kernel(
        _sc_gs_body,
        out_type=[jax.ShapeDtypeStruct((E_PAD, 64), jnp.float32),
                  jax.ShapeDtypeStruct((NC, N_PAD, 64), jnp.float32)],
        mesh=_mesh(),
        compiler_params=pltpu.CompilerParams(use_tc_tiling_on_sc=False),
        scratch_types=[pltpu.VMEM_SHARED((N_PAD, 64), jnp.float32),
                       pltpu.VMEM((CH,), jnp.int32),
                       pltpu.VMEM((CH, 64), jnp.float32),
                       pltpu.VMEM((CH, 64), jnp.float32),
                       pltpu.SemaphoreType.DMA])


def _sc_s_body(sidx, vals, zeros64, accp, acc, idx_v, bufs, sem):
    c, s, base = _worker_base()
    _acc_init(zeros64, acc, s)
    plsc.subcore_barrier()
    _scatter_loop(sidx, vals, acc, idx_v, bufs, base)
    plsc.subcore_barrier()
    _acc_dump(acc, accp, c, s)


@functools.lru_cache(maxsize=None)
def _sc_s():
    return pl.kernel(
        _sc_s_body,
        out_type=[jax.ShapeDtypeStruct((NC, N_PAD, 64), jnp.float32)],
        mesh=_mesh(),
        compiler_params=pltpu.CompilerParams(use_tc_tiling_on_sc=False),
        scratch_types=[pltpu.VMEM_SHARED((N_PAD, 64), jnp.float32),
                       pltpu.VMEM((CH,), jnp.int32),
                       pltpu.VMEM((CH, 64), jnp.float32),
                       pltpu.SemaphoreType.DMA])


RB = 1000
EB = 512
_NG = N // RB
_EG = E_PAD // EB


def _tc_nodeA_body(x_ref, w_ref, degp_ref, y_ref, dinv_ref):
    deg = degp_ref[0, :, 0:1] + degp_ref[1, :, 0:1] + 1.0
    dinv = lax.rsqrt(deg)
    xw = jnp.dot(x_ref[...], w_ref[...], preferred_element_type=jnp.float32)
    y_ref[...] = dinv * xw
    dinv_ref[...] = dinv


def _tc_msg1_body(eaT_ref, xs_ref, v0r_ref, v1r_ref, nbr_ref, out_ref):
    for g in range(EB // 128):
        lo = g * 128
        xsT = jnp.transpose(xs_ref[lo:lo + 128, :])
        a = eaT_ref[0:1, lo:lo + 128]
        b = eaT_ref[1:2, lo:lo + 128]
        acc = jnp.zeros((64, 128), jnp.float32)
        for i in range(8):
            r0 = i * 64
            z = (a * v0r_ref[r0:r0 + 64, :] + b * v1r_ref[r0:r0 + 64, :]
                 + nbr_ref[r0:r0 + 64, :])
            acc = acc + xsT[i:i + 1, :] * jnp.maximum(z, 0.0)
        out_ref[lo:lo + 128, :] = jnp.transpose(acc)


def _tc_h1_body(aggp_ref, xm_ref, r1_ref, b1_ref, out_ref):
    xr = jnp.dot(xm_ref[...], r1_ref[...], preferred_element_type=jnp.float32)
    out_ref[...] = aggp_ref[0] + aggp_ref[1] + xr + b1_ref[...]


def _tc_gmsg_body(ew_ref, g_ref, out_ref):
    out_ref[...] = ew_ref[...] * g_ref[...]


def _tc_msg2_body(eaT_ref, hs_ref, w0r_ref, w1r_ref, nbr_ref, out_ref):
    for gp in range(EB // 256):
        lo0 = gp * 256
        lo1 = lo0 + 128
        hsT0 = jnp.transpose(hs_ref[lo0:lo0 + 128, :])
        hsT1 = jnp.transpose(hs_ref[lo1:lo1 + 128, :])
        a0 = eaT_ref[0:1, lo0:lo0 + 128]
        b0 = eaT_ref[1:2, lo0:lo0 + 128]
        a1 = eaT_ref[0:1, lo1:lo1 + 128]
        b1 = eaT_ref[1:2, lo1:lo1 + 128]
        acc0 = jnp.zeros((64, 128), jnp.float32)
        acc1 = jnp.zeros((64, 128), jnp.float32)
        for i in range(64):
            r0 = i * 64
            w0t = w0r_ref[r0:r0 + 64, :]
            w1t = w1r_ref[r0:r0 + 64, :]
            nbt = nbr_ref[r0:r0 + 64, :]
            z0 = a0 * w0t + b0 * w1t + nbt
            z1 = a1 * w0t + b1 * w1t + nbt
            acc0 = acc0 + hsT0[i:i + 1, :] * jnp.maximum(z0, 0.0)
            acc1 = acc1 + hsT1[i:i + 1, :] * jnp.maximum(z1, 0.0)
        out_ref[lo0:lo0 + 128, :] = jnp.transpose(acc0)
        out_ref[lo1:lo1 + 128, :] = jnp.transpose(acc1)


def _tc_final_body(gaccp_ref, agg2p_ref, dinv_ref, y_ref, h1_ref, hid_ref,
                   x_ref, r2_ref, b2_ref, gb_ref, hA_ref, hB_ref, hC_ref,
                   hD_ref, hb_ref, sA_ref, sB_ref, sC_ref, sb_ref,
                   sc_ref, hs_ref):
    dinv = dinv_ref[...]
    epi = dinv * (gaccp_ref[0] + gaccp_ref[1]) + dinv * y_ref[...] + gb_ref[...]
    info = (agg2p_ref[0] + agg2p_ref[1]
            + jnp.dot(h1_ref[...], r2_ref[...],
                      preferred_element_type=jnp.float32)
            + b2_ref[...])
    t = (jnp.dot(epi, hA_ref[...], preferred_element_type=jnp.float32)
         + jnp.dot(info, hB_ref[...], preferred_element_type=jnp.float32)
         + jnp.dot(hid_ref[...], hC_ref[...], preferred_element_type=jnp.float32)
         + jnp.dot(x_ref[...], hD_ref[...], preferred_element_type=jnp.float32)
         + hb_ref[...])
    hs = jnp.maximum(t, 0.0)
    nrm = jnp.sqrt(jnp.sum(hs * hs, axis=1, keepdims=True))
    hsn = hs / jnp.maximum(nrm, 1e-12)
    s = (jnp.sum(hid_ref[...] * sA_ref[...], axis=1, keepdims=True)
         + jnp.sum(hsn * sB_ref[...], axis=1, keepdims=True)
         + jnp.sum(x_ref[...] * sC_ref[...], axis=1, keepdims=True)
         + sb_ref[...])
    sc_ref[...] = jnp.maximum(s, 0.0)
    hs_ref[...] = hsn


def _full(shape):
    return pl.BlockSpec(shape, lambda i: (0,) * len(shape))


def _rows(shape):
    return pl.BlockSpec(shape, lambda i: (i,) + (0,) * (len(shape) - 1))


def _mid(shape):
    return pl.BlockSpec(shape, lambda i: (0, i) + (0,) * (len(shape) - 2))


def kernel(x, edge_index, edge_weight, x_multi, edge_index_multi,
           edge_attr_multi, hidden_states, gcn_W, gcn_b, nn1_W, nn1_b,
           root1, b1, nn2_W, nn2_b, root2, b2, h_W, h_b, s_W, s_b):
    f32 = jnp.float32
    pe = E_PAD - E
    row = jnp.pad(edge_index[0], (0, pe))
    col = jnp.pad(edge_index[1], (0, pe), constant_values=N)
    srcm = jnp.pad(edge_index_multi[0], (0, pe))
    dstm = jnp.pad(edge_index_multi[1], (0, pe), constant_values=N)
    ew = jnp.pad(edge_weight, (0, pe))
    ea = jnp.pad(edge_attr_multi, ((0, pe), (0, 0)))
    ew16 = jnp.pad(ew[:, None], ((0, 0), (0, 15)))
    xm16 = jnp.pad(x_multi, ((0, 0), (0, 8)))
    zeros16 = jnp.zeros((N_PAD, 16), f32)
    zeros64 = jnp.zeros((N_PAD, 64), f32)

    eaT = jnp.transpose(ea)
    v0r = jnp.broadcast_to(nn1_W[0][:, None], (512, 128))
    v1r = jnp.broadcast_to(nn1_W[1][:, None], (512, 128))
    nb1r = jnp.broadcast_to(nn1_b[:, None], (512, 128))
    w0r = jnp.broadcast_to(nn2_W[0][:, None], (4096, 128))
    w1r = jnp.broadcast_to(nn2_W[1][:, None], (4096, 128))
    nb2r = jnp.broadcast_to(nn2_b[:, None], (4096, 128))
    b1r = b1.reshape(1, 64)
    b2r = b2.reshape(1, 64)
    gbr = gcn_b.reshape(1, 64)
    hA = h_W[0:64]
    hB = h_W[64:128]
    hC = h_W[128:192]
    hD = h_W[192:200]
    hbr = h_b.reshape(1, 64)
    sA = s_W[0:64].reshape(1, 64)
    sB = s_W[64:128].reshape(1, 64)
    sC = s_W[128:136].reshape(1, 8)
    sbr = s_b.reshape(1, 1)

    xs16, degp = _sc1()(srcm, xm16, col, ew16, zeros16)

    y, dinv = pl.pallas_call(
        _tc_nodeA_body,
        grid=(_NG,),
        in_specs=[_rows((RB, 8)), _full((8, 64)), _mid((NC, RB, 16))],
        out_specs=[_rows((RB, 64)), _rows((RB, 1))],
        out_shape=[jax.ShapeDtypeStruct((N, 64), f32),
                   jax.ShapeDtypeStruct((N, 1), f32)],
    )(x, gcn_W, degp)

    msg1 = pl.pallas_call(
        _tc_msg1_body,
        grid=(_EG,),
        in_specs=[pl.BlockSpec((2, EB), lambda i: (0, i)), _rows((EB, 16)),
                  _full((512, 128)), _full((512, 128)), _full((512, 128))],
        out_specs=_rows((EB, 64)),
        out_shape=jax.ShapeDtypeStruct((E_PAD, 64), f32),
    )(eaT, xs16, v0r, v1r, nb1r)

    g64, agg1p = _sc_gs()(row, y, dstm, msg1, zeros64)

    h1 = pl.pallas_call(
        _tc_h1_body,
        grid=(_NG,),
        in_specs=[_mid((NC, RB, 64)), _rows((RB, 8)), _full((8, 64)),
                  _full((1, 64))],
        out_specs=_rows((RB, 64)),
        out_shape=jax.ShapeDtypeStruct((N, 64), f32),
    )(agg1p, x_multi, root1, b1r)

    gmsg = pl.pallas_call(
        _tc_gmsg_body,
        grid=(_EG,),
        in_specs=[_rows((EB, 1)), _rows((EB, 64))],
        out_specs=_rows((EB, 64)),
        out_shape=jax.ShapeDtypeStruct((E_PAD, 64), f32),
    )(ew[:, None], g64)

    hs64, gaccp = _sc_gs()(srcm, h1, col, gmsg, zeros64)

    msg2 = pl.pallas_call(
        _tc_msg2_body,
        grid=(_EG,),
        in_specs=[pl.BlockSpec((2, EB), lambda i: (0, i)), _rows((EB, 64)),
                  _full((4096, 128)), _full((4096, 128)), _full((4096, 128))],
        out_specs=_rows((EB, 64)),
        out_shape=jax.ShapeDtypeStruct((E_PAD, 64), f32),
    )(eaT, hs64, w0r, w1r, nb2r)

    (agg2p,) = _sc_s()(dstm, msg2, zeros64)

    sc, hs = pl.pallas_call(
        _tc_final_body,
        grid=(_NG,),
        in_specs=[_mid((NC, RB, 64)), _mid((NC, RB, 64)), _rows((RB, 1)),
                  _rows((RB, 64)), _rows((RB, 64)), _rows((RB, 64)),
                  _rows((RB, 8)), _full((64, 64)), _full((1, 64)),
                  _full((1, 64)), _full((64, 64)), _full((64, 64)),
                  _full((64, 64)), _full((8, 64)), _full((1, 64)),
                  _full((1, 64)), _full((1, 64)), _full((1, 8)),
                  _full((1, 1))],
        out_specs=[_rows((RB, 1)), _rows((RB, 64))],
        out_shape=[jax.ShapeDtypeStruct((N, 1), f32),
                   jax.ShapeDtypeStruct((N, 64), f32)],
    )(gaccp, agg2p, dinv, y, h1, hidden_states, x, root2, b2r, gbr,
      hA, hB, hC, hD, hbr, sA, sB, sC, sbr)

    return (sc, hs)


# speedup vs baseline: 4.1447x; 2.1014x over previous
import functools

import jax
import jax.numpy as jnp
from jax import lax
from jax.experimental import pallas as pl
from jax.experimental.pallas import tpu as pltpu
from jax.experimental.pallas import tpu_sc as plsc

N = 10000
E = 160000
NC = 2
NS = 16
NW = NC * NS
CH = 128
EPW = 5120
E_PAD = NW * EPW
NCH = EPW // CH
N_PAD = 10240
NPT = N_PAD // NS

@functools.lru_cache(maxsize=None)
def _mesh():
    return plsc.VectorSubcoreMesh(
        core_axis_name="c", subcore_axis_name="s",
        num_cores=NC, num_subcores=NS)


def _worker_base():
    c = lax.axis_index("c")
    s = lax.axis_index("s")
    wid = s * NC + c
    return c, s, wid * EPW


K = 4


def _gather_loop(idx_hbm, table_hbm, out_hbm, idxs, bufs, isem, gsem, wsem,
                 base, kk=K, nch=NCH):
    def step(jo, carry):
        off0 = base + jo * (kk * CH)
        ic = [pltpu.async_copy(
            idx_hbm.at[pl.ds(pl.multiple_of(off0 + b * CH, CH), CH)],
            idxs.at[b], isem) for b in range(kk)]
        for d in ic:
            d.wait()
        gc = [pltpu.async_copy(table_hbm.at[idxs.at[b]], bufs.at[b], gsem)
              for b in range(kk)]
        for d in gc:
            d.wait()
        wc = [pltpu.async_copy(
            bufs.at[b], out_hbm.at[pl.ds(pl.multiple_of(off0 + b * CH, CH),
                                         CH)], wsem) for b in range(kk)]
        for d in wc:
            d.wait()
        return carry
    lax.fori_loop(0, nch // kk, step, 0)


def _scatter_loop(idx_hbm, vals_hbm, acc_shared, idxs, bufs, isem, vsem, ssem,
                  base, nch=NCH, kk=K):
    def step(jo, carry):
        off0 = base + jo * (kk * CH)
        ic = [pltpu.async_copy(
            idx_hbm.at[pl.ds(pl.multiple_of(off0 + b * CH, CH), CH)],
            idxs.at[b], isem) for b in range(kk)]
        vc = [pltpu.async_copy(
            vals_hbm.at[pl.ds(pl.multiple_of(off0 + b * CH, CH), CH)],
            bufs.at[b], vsem) for b in range(kk)]
        for d in ic + vc:
            d.wait()
        sc = [pltpu.async_copy(bufs.at[b], acc_shared.at[idxs.at[b]], ssem,
                               add=True) for b in range(kk)]
        for d in sc:
            d.wait()
        return carry
    lax.fori_loop(0, nch // kk, step, 0)


def _gcn_fused_loop(row_hbm, col_hbm, y_hbm, ew_hbm, acc_shared, idxs, bufs,
                    ewv, isem, vsem, gsem, ssem, base):
    def step(jo, carry):
        off0 = base + jo * (K * CH)
        ic = [pltpu.async_copy(
            row_hbm.at[pl.ds(pl.multiple_of(off0 + b * CH, CH), CH)],
            idxs.at[b], isem) for b in range(K)]
        wc = [pltpu.async_copy(
            ew_hbm.at[pl.ds(pl.multiple_of(off0 + b * CH, CH), CH)],
            ewv.at[b], vsem) for b in range(K)]
        for d in ic + wc:
            d.wait()
        gc = [pltpu.async_copy(y_hbm.at[idxs.at[b]], bufs.at[b], gsem)
              for b in range(K)]
        for d in gc:
            d.wait()

        for b in range(K):
            def scale(g, c, b=b):
                e0 = g * 16
                wvec = ewv[b, pl.ds(e0, 16)]
                for l in range(16):
                    w = wvec[l]
                    for q in range(4):
                        bufs[b, e0 + l, q * 16:(q + 1) * 16] = (
                            bufs[b, e0 + l, q * 16:(q + 1) * 16] * w)
                return c
            lax.fori_loop(0, CH // 16, scale, 0)
        ic2 = [pltpu.async_copy(
            col_hbm.at[pl.ds(pl.multiple_of(off0 + b * CH, CH), CH)],
            idxs.at[b], isem) for b in range(K)]
        for d in ic2:
            d.wait()
        sc = [pltpu.async_copy(bufs.at[b], acc_shared.at[idxs.at[b]], ssem,
                               add=True) for b in range(K)]
        for d in sc:
            d.wait()
        return carry
    lax.fori_loop(0, NCH // K, step, 0)


def _acc_init(zeros_hbm, acc_shared, s):
    r = pl.multiple_of(s * NPT, NPT)
    pltpu.sync_copy(zeros_hbm.at[pl.ds(r, NPT)], acc_shared.at[pl.ds(r, NPT)])


def _acc_dump(acc_shared, out_hbm, c, s):
    r = pl.multiple_of(s * NPT, NPT)
    pltpu.sync_copy(acc_shared.at[pl.ds(r, NPT)], out_hbm.at[c, pl.ds(r, NPT)])


def _sc1_body(srcm, xm16, col, ew16, zeros16,
              xs16, degp, acc, idxs, bufg, bufs, s0, s1, s2, s3):
    c, s, base = _worker_base()
    _acc_init(zeros16, acc, s)
    _gather_loop(srcm, xm16, xs16, idxs, bufg, s0, s1, s2, base)
    plsc.subcore_barrier()
    _scatter_loop(col, ew16, acc, idxs, bufs, s0, s1, s3, base)
    plsc.subcore_barrier()
    _acc_dump(acc, degp, c, s)


@functools.lru_cache(maxsize=None)
def _sc1():
    return pl.kernel(
        _sc1_body,
        out_type=[jax.ShapeDtypeStruct((E_PAD, 16), jnp.float32),
                  jax.ShapeDtypeStruct((NC, N_PAD, 16), jnp.float32)],
        mesh=_mesh(),
        compiler_params=pltpu.CompilerParams(use_tc_tiling_on_sc=False),
        scratch_types=[pltpu.VMEM_SHARED((N_PAD, 16), jnp.float32),
                       pltpu.VMEM((K, CH), jnp.int32),
                       pltpu.VMEM((K, CH, 16), jnp.float32),
                       pltpu.VMEM((K, CH, 16), jnp.float32),
                       pltpu.SemaphoreType.DMA, pltpu.SemaphoreType.DMA,
                       pltpu.SemaphoreType.DMA, pltpu.SemaphoreType.DMA])


def _sc2a_body(row, col, y, ew, zeros64,
               gaccp, accg, idxs, bufs, ewv, s0, s1, s2, s3):
    c, s, base = _worker_base()
    _acc_init(zeros64, accg, s)
    plsc.subcore_barrier()
    _gcn_fused_loop(row, col, y, ew, accg, idxs, bufs, ewv,
                    s0, s1, s2, s3, base)
    plsc.subcore_barrier()
    _acc_dump(accg, gaccp, c, s)


@functools.lru_cache(maxsize=None)
def _sc2a():
    return pl.kernel(
        _sc2a_body,
        out_type=[jax.ShapeDtypeStruct((NC, N_PAD, 64), jnp.float32)],
        mesh=_mesh(),
        compiler_params=pltpu.CompilerParams(use_tc_tiling_on_sc=False),
        scratch_types=[pltpu.VMEM_SHARED((N_PAD, 64), jnp.float32),
                       pltpu.VMEM((K, CH), jnp.int32),
                       pltpu.VMEM((K, CH, 64), jnp.float32),
                       pltpu.VMEM((K, CH), jnp.float32),
                       pltpu.SemaphoreType.DMA, pltpu.SemaphoreType.DMA,
                       pltpu.SemaphoreType.DMA, pltpu.SemaphoreType.DMA])


EPT2 = E_PAD // NS
NCH2 = EPT2 // CH


def _sc_h1_body(dstm, msg1, xr1b, srcm, dep, h1out, hs64,
                acc, idxs, bufs, s0, s1, s2, s3):
    c, s, base = _worker_base()
    _acc_init(xr1b, acc, s)
    plsc.subcore_barrier()
    base2 = s * EPT2
    _scatter_loop(dstm, msg1, acc, idxs, bufs, s0, s1, s3, base2, NCH2, 2)
    plsc.subcore_barrier()

    @pl.when(c == 0)
    def _():
        r = pl.multiple_of(s * NPT, NPT)
        pltpu.sync_copy(acc.at[pl.ds(r, NPT)], h1out.at[pl.ds(r, NPT)])

    _gather_loop(srcm, acc, hs64, idxs, bufs, s0, s1, s2, base, 2)


@functools.lru_cache(maxsize=None)
def _sc_h1():
    return pl.kernel(
        _sc_h1_body,
        out_type=[jax.ShapeDtypeStruct((N_PAD, 128), jnp.float32),
                  jax.ShapeDtypeStruct((E_PAD, 128), jnp.float32)],
        mesh=_mesh(),
        compiler_params=pltpu.CompilerParams(use_tc_tiling_on_sc=False),
        scratch_types=[pltpu.VMEM_SHARED((N_PAD, 128), jnp.float32),
                       pltpu.VMEM((2, CH), jnp.int32),
                       pltpu.VMEM((2, CH, 128), jnp.float32),
                       pltpu.SemaphoreType.DMA, pltpu.SemaphoreType.DMA,
                       pltpu.SemaphoreType.DMA, pltpu.SemaphoreType.DMA])


def _sc_s_body(sidx, vals, zeros64, accp, acc, idxs, bufs, s0, s1, s3):
    c, s, base = _worker_base()
    _acc_init(zeros64, acc, s)
    plsc.subcore_barrier()
    _scatter_loop(sidx, vals, acc, idxs, bufs, s0, s1, s3, base, NCH, 2)
    plsc.subcore_barrier()
    _acc_dump(acc, accp, c, s)


@functools.lru_cache(maxsize=None)
def _sc_s():
    return pl.kernel(
        _sc_s_body,
        out_type=[jax.ShapeDtypeStruct((NC, N_PAD, 128), jnp.float32)],
        mesh=_mesh(),
        compiler_params=pltpu.CompilerParams(use_tc_tiling_on_sc=False),
        scratch_types=[pltpu.VMEM_SHARED((N_PAD, 128), jnp.float32),
                       pltpu.VMEM((2, CH), jnp.int32),
                       pltpu.VMEM((2, CH, 128), jnp.float32),
                       pltpu.SemaphoreType.DMA, pltpu.SemaphoreType.DMA,
                       pltpu.SemaphoreType.DMA])


RB = 1000
EB = 512
_NG = N // RB
_EG = E_PAD // EB


def _tc_nodeA_body(x_ref, w_ref, degp_ref, xm_ref, r1_ref, b1_ref,
                   y_ref, dinv_ref, xr1b_ref):
    deg = degp_ref[0, :, 0:1] + degp_ref[1, :, 0:1] + 1.0
    dinv = lax.rsqrt(deg)
    xw = jnp.dot(x_ref[...], w_ref[...], preferred_element_type=jnp.float32)
    y_ref[...] = dinv * xw
    dinv_ref[...] = dinv
    xr1b_ref[:, 0:64] = (jnp.dot(xm_ref[...], r1_ref[...],
                                 preferred_element_type=jnp.float32)
                         + b1_ref[...])
    xr1b_ref[:, 64:128] = jnp.zeros((xr1b_ref.shape[0], 64), jnp.float32)


def _tc_msg1_body(eaT_ref, xs_ref, w_ref, out_ref):
    for g in range(EB // 128):
        lo = g * 128
        xsT = jnp.transpose(xs_ref[lo:lo + 128, :])
        ab1 = jnp.concatenate(
            [eaT_ref[0:2, lo:lo + 128], jnp.ones((1, 128), jnp.float32),
             jnp.zeros((5, 128), jnp.float32)], axis=0)
        z = jnp.dot(w_ref[...], ab1, preferred_element_type=jnp.float32)
        acc = jnp.zeros((64, 128), jnp.float32)
        for i in range(8):
            r0 = i * 64
            acc = acc + xsT[i:i + 1, :] * jnp.maximum(z[r0:r0 + 64, :], 0.0)
        out_ref[lo:lo + 128, 0:64] = jnp.transpose(acc)
        out_ref[lo:lo + 128, 64:128] = jnp.zeros((128, 64), jnp.float32)


def _tc_msg2_body(eaT_ref, hs_ref, w_ref, out_ref):
    for g in range(EB // 128):
        lo = g * 128
        hsT = jnp.transpose(hs_ref[lo:lo + 128, 0:64])
        ab1 = jnp.concatenate(
            [eaT_ref[0:2, lo:lo + 128], jnp.ones((1, 128), jnp.float32),
             jnp.zeros((5, 128), jnp.float32)], axis=0)
        z = jnp.dot(w_ref[...], ab1, preferred_element_type=jnp.float32)
        acc = jnp.zeros((64, 128), jnp.float32)
        for i in range(64):
            r0 = i * 64
            acc = acc + hsT[i:i + 1, :] * jnp.maximum(z[r0:r0 + 64, :], 0.0)
        out_ref[lo:lo + 128, 0:64] = jnp.transpose(acc)
        out_ref[lo:lo + 128, 64:128] = jnp.zeros((128, 64), jnp.float32)


def _tc_final_body(gaccp_ref, agg2p_ref, dinv_ref, y_ref, h1_ref, hid_ref,
                   x_ref, r2_ref, b2_ref, gb_ref, hA_ref, hB_ref, hC_ref,
                   hD_ref, hb_ref, sA_ref, sB_ref, sC_ref, sb_ref,
                   sc_ref, hs_ref):
    dinv = dinv_ref[...]
    epi = dinv * (gaccp_ref[0] + gaccp_ref[1]) + dinv * y_ref[...] + gb_ref[...]
    info = (agg2p_ref[0, :, 0:64] + agg2p_ref[1, :, 0:64]
            + jnp.dot(h1_ref[:, 0:64], r2_ref[...],
                      preferred_element_type=jnp.float32)
            + b2_ref[...])
    t = (jnp.dot(epi, hA_ref[...], preferred_element_type=jnp.float32)
         + jnp.dot(info, hB_ref[...], preferred_element_type=jnp.float32)
         + jnp.dot(hid_ref[...], hC_ref[...], preferred_element_type=jnp.float32)
         + jnp.dot(x_ref[...], hD_ref[...], preferred_element_type=jnp.float32)
         + hb_ref[...])
    hs = jnp.maximum(t, 0.0)
    nrm = jnp.sqrt(jnp.sum(hs * hs, axis=1, keepdims=True))
    hsn = hs / jnp.maximum(nrm, 1e-12)
    s = (jnp.sum(hid_ref[...] * sA_ref[...], axis=1, keepdims=True)
         + jnp.sum(hsn * sB_ref[...], axis=1, keepdims=True)
         + jnp.sum(x_ref[...] * sC_ref[...], axis=1, keepdims=True)
         + sb_ref[...])
    sc_ref[...] = jnp.maximum(s, 0.0)
    hs_ref[...] = hsn


def _full(shape):
    return pl.BlockSpec(shape, lambda i: (0,) * len(shape))


def _rows(shape):
    return pl.BlockSpec(shape, lambda i: (i,) + (0,) * (len(shape) - 1))


def _mid(shape):
    return pl.BlockSpec(shape, lambda i: (0, i) + (0,) * (len(shape) - 2))


def kernel(x, edge_index, edge_weight, x_multi, edge_index_multi,
           edge_attr_multi, hidden_states, gcn_W, gcn_b, nn1_W, nn1_b,
           root1, b1, nn2_W, nn2_b, root2, b2, h_W, h_b, s_W, s_b):
    f32 = jnp.float32
    pe = E_PAD - E
    row = jnp.pad(edge_index[0], (0, pe))
    col = jnp.pad(edge_index[1], (0, pe), constant_values=N)
    srcm = jnp.pad(edge_index_multi[0], (0, pe))
    dstm = jnp.pad(edge_index_multi[1], (0, pe), constant_values=N)
    ew = jnp.pad(edge_weight, (0, pe))
    ea = jnp.pad(edge_attr_multi, ((0, pe), (0, 0)))
    ew16 = jnp.pad(ew[:, None], ((0, 0), (0, 15)))
    xm16 = jnp.pad(x_multi, ((0, 0), (0, 8)))
    zeros16 = jnp.zeros((N_PAD, 16), f32)
    zeros64 = jnp.zeros((N_PAD, 64), f32)
    zeros128 = jnp.zeros((N_PAD, 128), f32)

    eaT = jnp.transpose(ea)
    wcat1 = jnp.concatenate(
        [nn1_W[0][:, None], nn1_W[1][:, None], nn1_b[:, None],
         jnp.zeros((512, 5), f32)], axis=1)
    wcat2 = jnp.concatenate(
        [nn2_W[0][:, None], nn2_W[1][:, None], nn2_b[:, None],
         jnp.zeros((4096, 5), f32)], axis=1)
    b1r = b1.reshape(1, 64)
    b2r = b2.reshape(1, 64)
    gbr = gcn_b.reshape(1, 64)
    hA = h_W[0:64]
    hB = h_W[64:128]
    hC = h_W[128:192]
    hD = h_W[192:200]
    hbr = h_b.reshape(1, 64)
    sA = s_W[0:64].reshape(1, 64)
    sB = s_W[64:128].reshape(1, 64)
    sC = s_W[128:136].reshape(1, 8)
    sbr = s_b.reshape(1, 1)

    xs16, degp = _sc1()(srcm, xm16, col, ew16, zeros16)

    x_pad = jnp.pad(x, ((0, N_PAD - N), (0, 0)))
    xm_pad = jnp.pad(x_multi, ((0, N_PAD - N), (0, 0)))
    RB2 = 1024
    y, dinv, xr1b = pl.pallas_call(
        _tc_nodeA_body,
        grid=(N_PAD // RB2,),
        in_specs=[_rows((RB2, 8)), _full((8, 64)), _mid((NC, RB2, 16)),
                  _rows((RB2, 8)), _full((8, 64)), _full((1, 64))],
        out_specs=[_rows((RB2, 64)), _rows((RB2, 1)), _rows((RB2, 128))],
        out_shape=[jax.ShapeDtypeStruct((N_PAD, 64), f32),
                   jax.ShapeDtypeStruct((N_PAD, 1), f32),
                   jax.ShapeDtypeStruct((N_PAD, 128), f32)],
    )(x_pad, gcn_W, degp, xm_pad, root1, b1r)

    msg1 = pl.pallas_call(
        _tc_msg1_body,
        grid=(_EG,),
        in_specs=[pl.BlockSpec((2, EB), lambda i: (0, i)), _rows((EB, 16)),
                  _full((512, 8))],
        out_specs=_rows((EB, 128)),
        out_shape=jax.ShapeDtypeStruct((E_PAD, 128), f32),
    )(eaT, xs16, wcat1)

    (gaccp,) = _sc2a()(row, col, y, ew, zeros64)
    h1, hs64 = _sc_h1()(dstm, msg1, xr1b, srcm, gaccp)

    msg2 = pl.pallas_call(
        _tc_msg2_body,
        grid=(_EG,),
        in_specs=[pl.BlockSpec((2, EB), lambda i: (0, i)), _rows((EB, 128)),
                  _full((4096, 8))],
        out_specs=_rows((EB, 128)),
        out_shape=jax.ShapeDtypeStruct((E_PAD, 128), f32),
    )(eaT, hs64, wcat2)

    (agg2p,) = _sc_s()(dstm, msg2, zeros128)

    sc, hs = pl.pallas_call(
        _tc_final_body,
        grid=(_NG,),
        in_specs=[_mid((NC, RB, 64)), _mid((NC, RB, 128)), _rows((RB, 1)),
                  _rows((RB, 64)), _rows((RB, 128)), _rows((RB, 64)),
                  _rows((RB, 8)), _full((64, 64)), _full((1, 64)),
                  _full((1, 64)), _full((64, 64)), _full((64, 64)),
                  _full((64, 64)), _full((8, 64)), _full((1, 64)),
                  _full((1, 64)), _full((1, 64)), _full((1, 8)),
                  _full((1, 1))],
        out_specs=[_rows((RB, 1)), _rows((RB, 64))],
        out_shape=[jax.ShapeDtypeStruct((N, 1), f32),
                   jax.ShapeDtypeStruct((N, 64), f32)],
    )(gaccp, agg2p, dinv, y, h1, hidden_states, x, root2, b2r, gbr,
      hA, hB, hC, hD, hbr, sA, sB, sC, sbr)

    return (sc, hs)
